# baseline (speedup 1.0000x reference)
import functools

import jax
import jax.numpy as jnp
from jax import lax
from jax.experimental import pallas as pl
from jax.experimental.pallas import tpu as pltpu
from jax.experimental.pallas import tpu_sc as plsc

N = 10000
E = 320000
H = 8
C = 16
HC = H * C
ROW = HC + 16
NC = 2
NS = 16
NT = NC * NS
EPT = E // NT
B = 16
NBLK = EPT // B
ND = 4
NI = 8
MAIN = (NBLK // NI) * NI - NI
NPAD = 10240
RPS = NPAD // NS
SCALE = 1.0 / (C ** 0.5)

_mesh = plsc.VectorSubcoreMesh(core_axis_name="c", subcore_axis_name="s")
_params = pltpu.CompilerParams(needs_layout_passes=False,
                               use_tc_tiling_on_sc=False)

_DNUMS = lax.GatherDimensionNumbers(
    offset_dims=(), collapsed_slice_dims=(0,), start_index_map=(0,))


def _take16(x, idx):
    return lax.gather(x, idx[:, None], _DNUMS, (1,),
                      mode=lax.GatherScatterMode.PROMISE_IN_BOUNDS)


@functools.partial(
    pl.kernel,
    out_type=jax.ShapeDtypeStruct((NC, NPAD, ROW), jnp.float32),
    mesh=_mesh,
    compiler_params=_params,
    scratch_types=[
        pltpu.VMEM((NI, B), jnp.int32),
        pltpu.VMEM((NI, B), jnp.int32),
        pltpu.VMEM((ND, B, HC), jnp.float32),
        pltpu.VMEM((ND, B, 2 * HC), jnp.float32),
        pltpu.VMEM((ND, B * HC), jnp.float32),
        pltpu.VMEM((B, ROW), jnp.float32),
        pltpu.SemaphoreType.DMA,
        pltpu.SemaphoreType.DMA,
        pltpu.SemaphoreType.DMA,
        pltpu.SemaphoreType.DMA,
        pltpu.SemaphoreType.DMA,
        pltpu.VMEM_SHARED((NPAD, ROW), jnp.float32),
    ],
)
def _attn(qh, kvh, eh, srch, dsth, acch,
          sidx, didx, qb, kvb, eb, ob,
          semi, semd0, semd1, semd2, semd3, acc_sh):
    c = lax.axis_index("c")
    s = lax.axis_index("s")
    wid = c * NS + s
    semd = (semd0, semd1, semd2, semd3)

    lanes = lax.iota(jnp.int32, 16)
    tidx = [(lanes & 7) * C + 2 * j + (lanes >> 3) for j in range(H)]
    swap8 = lanes ^ 8
    bcast = [jnp.full((16,), j, jnp.int32) for j in range(H)]
    zero16 = jnp.zeros((16,), jnp.float32)

    def zrow(i, _):
        for j in range(ROW // 16):
            ob[i, pl.ds(16 * j, 16)] = zero16
        return 0
    lax.fori_loop(0, B, zrow, 0, unroll=False)

    def zcopy(u, _):
        pltpu.sync_copy(ob, acc_sh.at[pl.ds(s * RPS + u * B, B)])
        return 0
    lax.fori_loop(0, RPS // B, zcopy, 0, unroll=False)
    plsc.subcore_barrier()

    row_t = wid * NBLK

    def fire_idx(b, islot):
        pltpu.async_copy(srch.at[row_t + b], sidx.at[islot], semi)
        pltpu.async_copy(dsth.at[row_t + b], didx.at[islot], semi)

    def drain_idx(b, islot):
        pltpu.make_async_copy(srch.at[row_t + b], sidx.at[islot], semi).wait()
        pltpu.make_async_copy(dsth.at[row_t + b], didx.at[islot], semi).wait()

    def fire_gath(b, islot, slot):
        base = wid * EPT + b * B
        sem = semd[slot]
        pltpu.async_copy(eh.at[pl.ds(base * HC, B * HC)], eb.at[slot], sem)
        pltpu.async_copy(kvh.at[sidx.at[islot]], kvb.at[slot], sem)
        pltpu.async_copy(qh.at[didx.at[islot]], qb.at[slot], sem)

    def drain_gath(b, islot, slot):
        base = wid * EPT + b * B
        sem = semd[slot]
        pltpu.make_async_copy(eh.at[pl.ds(base * HC, B * HC)], eb.at[slot], sem).wait()
        pltpu.make_async_copy(kvh.at[sidx.at[islot]], kvb.at[slot], sem).wait()
        pltpu.make_async_copy(qh.at[didx.at[islot]], qb.at[slot], sem).wait()

    def compute(b, slot, par):
        def edge(i, _):
            rowv = jnp.full((16,), i, jnp.int32)
            rowe = rowv * HC
            acc0 = zero16
            acc1 = zero16
            for j in range(H):
                qt = plsc.load_gather(qb.at[slot], [rowv, tidx[j]])
                kt = plsc.load_gather(kvb.at[slot], [rowv, tidx[j]])
                et = plsc.load_gather(eb.at[slot], [rowe + tidx[j]])
                if j % 2 == 0:
                    acc0 = acc0 + qt * (kt + et)
                else:
                    acc1 = acc1 + qt * (kt + et)
            acc = acc0 + acc1
            acc = acc + _take16(acc, swap8)
            w = jnp.exp(acc * SCALE)
            ibase = i * HC
            vjs = [kvb[slot, i, pl.ds(HC + C * j, C)] for j in range(H)]
            ejs = [eb[slot, pl.ds(ibase + C * j, C)] for j in range(H)]
            wjs = [_take16(w, bcast[j]) for j in range(H)]
            for j in range(H):
                ob[i, pl.ds(C * j, C)] = wjs[j] * (vjs[j] + ejs[j])
            ob[i, pl.ds(HC, 16)] = w
            return 0
        lax.fori_loop(0, B, edge, 0, unroll=False)

    def step(b, pb, fire3, fire4, wait2):
        if fire3:
            drain_idx(b + 3, (pb + 3) % NI)
            fire_gath(b + 3, (pb + 3) % NI, (pb + 3) % ND)
        drain_gath(b, pb % NI, pb % ND)
        compute(b, pb % ND, pb % 2)
        pltpu.sync_copy(ob, acc_sh.at[didx.at[pb % NI]], add=True)
        if fire4:
            fire_idx(b + 4, (pb + 4) % NI)

    for b in range(4):
        fire_idx(b, b)
    for b in range(3):
        drain_idx(b, b)
        fire_gath(b, b, b)

    @pl.loop(0, MAIN, step=NI)
    def _blk(b0):
        for off in range(NI):
            step(b0 + off, off, fire3=True, fire4=True, wait2=True)

    for bb in range(MAIN, NBLK):
        step(bb, bb % NI,
             fire3=(bb + 3 < NBLK), fire4=(bb + 4 < NBLK),
             wait2=(bb - 2 >= 0))

    plsc.subcore_barrier()
    pltpu.sync_copy(acc_sh.at[pl.ds(s * RPS, RPS)],
                    acch.at[c, pl.ds(s * RPS, RPS)])


U = 40
NU = N // U


@functools.partial(
    pl.kernel,
    out_type=jax.ShapeDtypeStruct((N, HC), jnp.float32),
    mesh=_mesh,
    compiler_params=_params,
    scratch_types=[
        pltpu.VMEM((U, ROW), jnp.float32),
        pltpu.VMEM((U, ROW), jnp.float32),
        pltpu.VMEM((U, HC), jnp.float32),
    ],
)
def _finish(acch, outh, a0, a1, ob):
    c = lax.axis_index("c")
    s = lax.axis_index("s")
    wid = c * NS + s
    nu = jnp.where(wid < 26, 8, 7)
    ubase = jnp.where(wid < 26, wid * 8, 208 + (wid - 26) * 7)
    bcast = [jnp.full((16,), j, jnp.int32) for j in range(H)]

    def unit(u, _):
        r0 = (ubase + u) * U
        pltpu.sync_copy(acch.at[0, pl.ds(r0, U)], a0)
        pltpu.sync_copy(acch.at[1, pl.ds(r0, U)], a1)

        def row(i, _):
            d = a0[i, pl.ds(HC, 16)] + a1[i, pl.ds(HC, 16)]
            for j in range(H):
                nj = a0[i, pl.ds(C * j, C)] + a1[i, pl.ds(C * j, C)]
                dj = _take16(d, bcast[j])
                ob[i, pl.ds(C * j, C)] = nj / jnp.maximum(dj, 1e-30)
            return 0
        lax.fori_loop(0, U, row, 0, unroll=False)
        pltpu.sync_copy(ob, outh.at[pl.ds(r0, U)])
        return 0
    lax.fori_loop(0, nu, unit, 0, unroll=False)


def kernel(q, k, v, e, edge_index):
    q2 = q.reshape(N, HC)
    kv = jnp.concatenate([k.reshape(N, HC), v.reshape(N, HC)], axis=1)
    e1 = e.reshape(E * HC)
    src_i = edge_index[0].reshape(E // B, B)
    dst_i = edge_index[1].reshape(E // B, B)
    acc = _attn(q2, kv, e1, src_i, dst_i)
    out = _finish(acc)
    return out.reshape(N, H, C)
